# Initial kernel scaffold; baseline (speedup 1.0000x reference)
import jax
import jax.numpy as jnp
from jax.experimental import pallas as pl


def kernel(x, edge_index, batch, Wl0, Wr0, bl0, Wl1, Wr1, bl1, Wp1, bp1, Wp2, bp2):
    raise NotImplementedError("write your pallas kernel here")


# speedup vs baseline: 4.2395x; 4.2395x over previous
import functools

import jax
import jax.numpy as jnp
from jax import lax
from jax.experimental import pallas as pl
from jax.experimental.pallas import tpu as pltpu
from jax.experimental.pallas import tpu_sc as plsc

NC = 2
NS = 16
NW = NC * NS
LANES = 16
CHUNK = 128


def _make_sc_pass(n_pad, e_worker, d, with_deg):
    n_chunks = e_worker // CHUNK
    rows_per_sub = n_pad // NS
    mesh = plsc.VectorSubcoreMesh(core_axis_name="c", subcore_axis_name="s")

    out_type = [jax.ShapeDtypeStruct((NC * n_pad, d), jnp.float32)]
    scratch = [
        pltpu.VMEM((CHUNK,), jnp.int32),
        pltpu.VMEM((CHUNK,), jnp.int32),
        pltpu.VMEM((CHUNK, d), jnp.float32),
        pltpu.VMEM_SHARED((n_pad, d), jnp.float32),
    ]
    if with_deg:
        out_type.append(jax.ShapeDtypeStruct((NW * n_pad,), jnp.float32))
        scratch.append(pltpu.VMEM((n_pad,), jnp.float32))

    @functools.partial(
        pl.kernel, out_type=out_type, mesh=mesh, scratch_types=scratch,
        compiler_params=pltpu.CompilerParams(needs_layout_passes=False))
    def sc_pass(*refs):
        if with_deg:
            (src_hbm, dst_hbm, x_hbm, zero_hbm,
             agg_out, deg_out, src_v, dst_v, rows_v, acc_sh, deg_v) = refs
        else:
            (src_hbm, dst_hbm, x_hbm, zero_hbm,
             agg_out, src_v, dst_v, rows_v, acc_sh) = refs

        core = lax.axis_index("c")
        sid = lax.axis_index("s")
        wid = sid * NC + core
        rbase = sid * rows_per_sub

        pltpu.sync_copy(zero_hbm.at[pl.ds(rbase, rows_per_sub)],
                        acc_sh.at[pl.ds(rbase, rows_per_sub)])
        if with_deg:
            @pl.loop(0, n_pad // LANES)
            def _(i):
                deg_v[pl.ds(i * LANES, LANES)] = jnp.zeros(
                    (LANES,), jnp.float32)
        plsc.subcore_barrier()

        ebase = wid * e_worker

        @pl.loop(0, n_chunks)
        def _(ci):
            off = ebase + ci * CHUNK
            pltpu.sync_copy(src_hbm.at[pl.ds(off, CHUNK)], src_v)
            pltpu.sync_copy(dst_hbm.at[pl.ds(off, CHUNK)], dst_v)
            pltpu.sync_copy(x_hbm.at[src_v], rows_v)
            pltpu.sync_copy(rows_v, acc_sh.at[dst_v], add=True)
            if with_deg:
                @pl.loop(0, CHUNK // LANES)
                def _(j):
                    dv = dst_v[pl.ds(j * LANES, LANES)]
                    plsc.addupdate_scatter(
                        deg_v, [dv], jnp.full((LANES,), 1.0, jnp.float32))

        plsc.subcore_barrier()

        obase = core * n_pad + rbase
        pltpu.sync_copy(acc_sh.at[pl.ds(rbase, rows_per_sub)],
                        agg_out.at[pl.ds(obase, rows_per_sub)])
        if with_deg:
            pltpu.sync_copy(deg_v, deg_out.at[pl.ds(wid * n_pad, n_pad)])

    return sc_pass


def _tc_sage_layer(agg, degp, x, wlT, wrT, bl, blk):
    n, d = x.shape

    def body(agg0_r, agg1_r, degp_r, x_r, wl_r, wr_r, bl_r, o_r, deg_r):
        deg = jnp.maximum(jnp.sum(degp_r[...], axis=1, keepdims=True), 1.0)
        mean = (agg0_r[0] + agg1_r[0]) / deg
        h = (jnp.dot(mean, wl_r[...], precision=lax.Precision.HIGHEST,
                     preferred_element_type=jnp.float32)
             + bl_r[...]
             + jnp.dot(x_r[...], wr_r[...], precision=lax.Precision.HIGHEST,
                       preferred_element_type=jnp.float32))
        o_r[...] = jnp.maximum(h, 0.0)
        deg_r[...] = deg

    return pl.pallas_call(
        body,
        grid=(n // blk,),
        in_specs=[
            pl.BlockSpec((1, blk, d), lambda i: (0, i, 0)),
            pl.BlockSpec((1, blk, d), lambda i: (1, i, 0)),
            pl.BlockSpec((blk, NW), lambda i: (i, 0)),
            pl.BlockSpec((blk, d), lambda i: (i, 0)),
            pl.BlockSpec((d, d), lambda i: (0, 0)),
            pl.BlockSpec((d, d), lambda i: (0, 0)),
            pl.BlockSpec((1, d), lambda i: (0, 0)),
        ],
        out_specs=[
            pl.BlockSpec((blk, d), lambda i: (i, 0)),
            pl.BlockSpec((blk, 1), lambda i: (i, 0)),
        ],
        out_shape=[
            jax.ShapeDtypeStruct((n, d), jnp.float32),
            jax.ShapeDtypeStruct((n, 1), jnp.float32),
        ],
    )(agg, agg, degp, x, wlT, wrT, bl.reshape(1, d))


def _tc_layer1_post(agg, deg, h, wlT, wrT, bl, wp1T, bp1, wp2T, bp2, blk):
    n, d = h.shape
    d_mid = wp1T.shape[1]
    d_out = wp2T.shape[1]

    def body(agg0_r, agg1_r, deg_r, h_r, wl_r, wr_r, bl_r,
             wp1_r, bp1_r, wp2_r, bp2_r, o_r):
        mean = (agg0_r[0] + agg1_r[0]) / deg_r[...]
        h1 = (jnp.dot(mean, wl_r[...], precision=lax.Precision.HIGHEST,
                      preferred_element_type=jnp.float32)
              + bl_r[...]
              + jnp.dot(h_r[...], wr_r[...], precision=lax.Precision.HIGHEST,
                        preferred_element_type=jnp.float32))
        h1 = jnp.maximum(h1, 0.0)
        t = jnp.maximum(
            jnp.dot(h1, wp1_r[...], precision=lax.Precision.HIGHEST,
                    preferred_element_type=jnp.float32) + bp1_r[...], 0.0)
        z = jnp.dot(t, wp2_r[...], precision=lax.Precision.HIGHEST,
                    preferred_element_type=jnp.float32) + bp2_r[...]
        m = jnp.max(z, axis=1, keepdims=True)
        lse = m + jnp.log(jnp.sum(jnp.exp(z - m), axis=1, keepdims=True))
        o_r[...] = z - lse

    return pl.pallas_call(
        body,
        grid=(n // blk,),
        in_specs=[
            pl.BlockSpec((1, blk, d), lambda i: (0, i, 0)),
            pl.BlockSpec((1, blk, d), lambda i: (1, i, 0)),
            pl.BlockSpec((blk, 1), lambda i: (i, 0)),
            pl.BlockSpec((blk, d), lambda i: (i, 0)),
            pl.BlockSpec((d, d), lambda i: (0, 0)),
            pl.BlockSpec((d, d), lambda i: (0, 0)),
            pl.BlockSpec((1, d), lambda i: (0, 0)),
            pl.BlockSpec((d, d_mid), lambda i: (0, 0)),
            pl.BlockSpec((1, d_mid), lambda i: (0, 0)),
            pl.BlockSpec((d_mid, d_out), lambda i: (0, 0)),
            pl.BlockSpec((1, d_out), lambda i: (0, 0)),
        ],
        out_specs=pl.BlockSpec((blk, d_out), lambda i: (i, 0)),
        out_shape=jax.ShapeDtypeStruct((n, d_out), jnp.float32),
    )(agg, agg, deg, h, wlT, wrT, bl.reshape(1, d),
      wp1T, bp1.reshape(1, d_mid), wp2T, bp2.reshape(1, d_out))


def kernel(x, edge_index, batch, Wl0, Wr0, bl0, Wl1, Wr1, bl1,
           Wp1, bp1, Wp2, bp2):
    n, d = x.shape
    e = edge_index.shape[1]

    e_worker = -(-e // (NW * CHUNK)) * CHUNK
    e_pad = NW * e_worker
    n_pad = -(-n // (NS * 8)) * (NS * 8)
    if n_pad == n:
        n_pad = n + NS * 8

    src = edge_index[0]
    dst = edge_index[1]
    pad = e_pad - e
    srcp = jnp.concatenate([src, jnp.zeros((pad,), jnp.int32)])
    dstp = jnp.concatenate([dst, jnp.full((pad,), n_pad - 1, jnp.int32)])
    zf = jnp.zeros((n_pad, d), jnp.float32)

    sc_pass0 = _make_sc_pass(n_pad, e_worker, d, with_deg=True)
    sc_pass1 = _make_sc_pass(n_pad, e_worker, d, with_deg=False)

    agg0, degp = sc_pass0(srcp, dstp, x, zf)
    agg0 = agg0.reshape(NC, n_pad, d)
    degp = degp.reshape(NW, n_pad).T
    h0, deg = _tc_sage_layer(agg0, degp, x, Wl0.T, Wr0.T, bl0, blk=2000)
    (agg1,) = sc_pass1(srcp, dstp, h0, zf)
    agg1 = agg1.reshape(NC, n_pad, d)
    return _tc_layer1_post(agg1, deg, h0, Wl1.T, Wr1.T, bl1,
                           Wp1.T, bp1, Wp2.T, bp2, blk=2000)
